# baseline (speedup 1.0000x reference)
import functools

import jax
import jax.numpy as jnp
from jax import lax
from jax.experimental import pallas as pl
from jax.experimental.pallas import tpu as pltpu
from jax.experimental.pallas import tpu_sc as plsc

T = 2048
N = 1024
BH = 32
TB = 512
N_TB = T // TB

_NC, _NS = 2, 16
_NW = _NC * _NS
_B_PER_W = T // _NW


@functools.cache
def _make_sc_gather():
    mesh = plsc.VectorSubcoreMesh(core_axis_name="c", subcore_axis_name="s")

    @functools.partial(
        pl.kernel,
        out_type=jax.ShapeDtypeStruct((T, N), jnp.float32),
        mesh=mesh,
        scratch_types=[
            pltpu.VMEM((_B_PER_W,), jnp.int32),
            pltpu.VMEM((_B_PER_W, N), jnp.float32),
            pltpu.SemaphoreType.DMA,
        ],
    )
    def _sc_gather(idx_hbm, table_hbm, out_hbm, idx_v, rows_v, sem):
        wid = lax.axis_index("s") * _NC + lax.axis_index("c")
        base = wid * _B_PER_W
        pltpu.sync_copy(idx_hbm.at[pl.ds(base, _B_PER_W)], idx_v)
        pltpu.async_copy(table_hbm.at[idx_v], rows_v, sem).wait()
        pltpu.sync_copy(rows_v, out_hbm.at[pl.ds(base, _B_PER_W)])

    return _sc_gather


def _fused_body(x_ref, e_ref, w_ref, b_ref, o_ref, te_ref):
    bh = pl.program_id(1)

    @pl.when(bh == 0)
    def _project():
        te_ref[...] = (
            lax.dot_general(
                e_ref[...], w_ref[...],
                (((1,), (1,)), ((), ())),
                preferred_element_type=jnp.float32,
            )
            + b_ref[...]
        )

    o_ref[...] = x_ref[...] + te_ref[...][None]


def _fused_call(xr, e, W, b2):
    return pl.pallas_call(
        _fused_body,
        grid=(N_TB, BH),
        in_specs=[
            pl.BlockSpec((1, TB, N), lambda tb, bh: (bh, tb, 0)),
            pl.BlockSpec((TB, N), lambda tb, bh: (tb, 0)),
            pl.BlockSpec((N, N), lambda tb, bh: (0, 0)),
            pl.BlockSpec((1, N), lambda tb, bh: (0, 0)),
        ],
        out_specs=pl.BlockSpec((1, TB, N), lambda tb, bh: (bh, tb, 0)),
        out_shape=jax.ShapeDtypeStruct((BH, T, N), jnp.float32),
        scratch_shapes=[pltpu.VMEM((TB, N), jnp.float32)],
    )(xr, e, W, b2)


def kernel(x, t, emb, W, b):
    e = _make_sc_gather()(t, emb)
    xr = x.reshape(BH, T, N)
    out = _fused_call(xr, e, W, b.reshape(1, N))
    return out.reshape(x.shape)
